# Initial kernel scaffold; baseline (speedup 1.0000x reference)
import jax
import jax.numpy as jnp
from jax.experimental import pallas as pl


def kernel(xyz, W0_0, b0_0, W0_1, b0_1, W0_2, b0_2, W1_0, b1_0, W1_1, b1_1, W1_2, b1_2, W2_0, b2_0, W2_1, b2_1, W2_2, b2_2, Wp, bp):
    raise NotImplementedError("write your pallas kernel here")


# speedup vs baseline: 18.3709x; 18.3709x over previous
import functools

import jax
import jax.numpy as jnp
from jax import lax
from jax.experimental import pallas as pl
from jax.experimental.pallas import tpu as pltpu
from jax.experimental.pallas import tpu_sc as plsc

B = 16
N = 2048
S = 256
RADII = (0.1, 0.2, 0.4)
NSAMPLES = (16, 32, 128)
SH = 128


def _fps_body(xt_ref, cent_ref):
    x = xt_ref[0]
    y = xt_ref[1]
    z = xt_ref[2]
    iota_n = lax.broadcasted_iota(jnp.int32, (B, N), 1)
    iota_s = lax.broadcasted_iota(jnp.int32, (B, S), 1)

    def body(i, state):
        dist, far, ncx, ncy, ncz = state
        onehot = iota_n == far
        cx = jnp.sum(jnp.where(onehot, x, 0.0), axis=1, keepdims=True)
        cy = jnp.sum(jnp.where(onehot, y, 0.0), axis=1, keepdims=True)
        cz = jnp.sum(jnp.where(onehot, z, 0.0), axis=1, keepdims=True)
        sel = iota_s == i
        ncx = jnp.where(sel, cx, ncx)
        ncy = jnp.where(sel, cy, ncy)
        ncz = jnp.where(sel, cz, ncz)
        d = (x - cx) ** 2 + (y - cy) ** 2 + (z - cz) ** 2
        dist = jnp.minimum(dist, d)
        m = jnp.max(dist, axis=1, keepdims=True)
        far = jnp.min(jnp.where(dist == m, iota_n, N), axis=1, keepdims=True)
        return dist, far, ncx, ncy, ncz

    dist0 = jnp.full((B, N), 1e10, jnp.float32)
    far0 = jnp.zeros((B, 1), jnp.int32)
    z0 = jnp.zeros((B, S), jnp.float32)
    _, _, ncx, ncy, ncz = lax.fori_loop(0, S, body, (dist0, far0, z0, z0, z0))
    cent_ref[0] = ncx
    cent_ref[1] = ncy
    cent_ref[2] = ncz


def _run_fps(xt):
    return pl.pallas_call(
        _fps_body,
        out_shape=jax.ShapeDtypeStruct((3, B, S), jnp.float32),
    )(xt)


def _mask_body(ca_ref, xt_ref, code_ref):
    ca = ca_ref[0]
    xb = xt_ref[0]
    m = jnp.dot(ca, xb, preferred_element_type=jnp.float32)
    cn = jnp.sum(ca * ca, axis=1, keepdims=True)
    pn = jnp.sum(xb * xb, axis=0, keepdims=True)
    sqr = -2.0 * m
    sqr = sqr + cn
    sqr = sqr + pn
    code = jnp.where(sqr <= RADII[0] * RADII[0], 1, 0)
    code = code + jnp.where(sqr <= RADII[1] * RADII[1], 2, 0)
    code = code + jnp.where(sqr <= RADII[2] * RADII[2], 4, 0)
    code_ref[0] = code


def _run_mask(ca, xtb):
    return pl.pallas_call(
        _mask_body,
        grid=(B,),
        in_specs=[
            pl.BlockSpec((1, S, 3), lambda i: (i, 0, 0)),
            pl.BlockSpec((1, 3, N), lambda i: (i, 0, 0)),
        ],
        out_specs=pl.BlockSpec((1, S, N), lambda i: (i, 0, 0)),
        out_shape=jax.ShapeDtypeStruct((B, S, N), jnp.int32),
    )(ca, xtb)


def _group_body(xt_hbm, cent_hbm, code_hbm, g0_hbm, g1_hbm, g2_hbm,
                xs_v, ys_v, zs_v, cent_v, crow_v, ib0, ib1, ib2,
                ob0, ob1, ob2):
    cid = lax.axis_index("c")
    sid = lax.axis_index("s")
    b = sid
    s0 = cid * SH

    pltpu.sync_copy(xt_hbm.at[0, b], xs_v)
    pltpu.sync_copy(xt_hbm.at[1, b], ys_v)
    pltpu.sync_copy(xt_hbm.at[2, b], zs_v)
    pltpu.sync_copy(cent_hbm.at[0, b, pl.ds(s0, SH)], cent_v.at[pl.ds(0, SH)])
    pltpu.sync_copy(cent_hbm.at[1, b, pl.ds(s0, SH)], cent_v.at[pl.ds(SH, SH)])
    pltpu.sync_copy(cent_hbm.at[2, b, pl.ds(s0, SH)], cent_v.at[pl.ds(2 * SH, SH)])

    lanes = lax.iota(jnp.int32, 16)
    zero16 = jnp.zeros((16,), jnp.int32)
    scales = ((NSAMPLES[0], ib0, ob0),
              (NSAMPLES[1], ib1, ob1),
              (NSAMPLES[2], ib2, ob2))

    def run_s(s, _):
        pltpu.sync_copy(code_hbm.at[b, s0 + s], crow_v)
        sv = jnp.zeros((16,), jnp.int32) + s
        cx = plsc.load_gather(cent_v, [sv])
        cy = plsc.load_gather(cent_v, [sv + SH])
        cz = plsc.load_gather(cent_v, [sv + 2 * SH])

        def scan_chunk(c, state):
            cnts, mns = state
            base = c * 16
            cv = crow_v[pl.ds(base, 16)]
            inds = base + lanes
            new_cnts = []
            new_mns = []
            for bit, ((nsamp, ib, ob), cn, mn) in enumerate(
                    zip(scales, cnts, mns)):
                msk = (cv & (1 << bit)) != 0
                csum = plsc.cumsum(jnp.where(msk, 1, 0))
                pos = cn + csum - 1
                wmask = msk & (pos < nsamp)
                plsc.store_scatter(ib, [pos], inds, mask=wmask)
                new_cnts.append(cn + plsc.all_reduce_population_count(msk))
                new_mns.append(jnp.minimum(mn, jnp.where(msk, inds, 1 << 30)))
            return tuple(new_cnts), tuple(new_mns)

        big16 = jnp.zeros((16,), jnp.int32) + (1 << 30)
        cnts, mns = lax.fori_loop(
            0, N // 16, scan_chunk,
            ((zero16, zero16, zero16), (big16, big16, big16)))

        for (nsamp, ib, ob), cn, mn in zip(scales, cnts, mns):
            fs = jnp.min(mn)
            first = jnp.minimum(jnp.zeros((16,), jnp.int32) + fs, N - 1)
            obase = s * nsamp * 3
            for j in range(nsamp // 16):
                posj = j * 16 + lanes
                iv = ib[pl.ds(j * 16, 16)]
                ivf = jnp.where(posj < cn, iv, first)
                gx = plsc.load_gather(xs_v, [ivf]) - cx
                gy = plsc.load_gather(ys_v, [ivf]) - cy
                gz = plsc.load_gather(zs_v, [ivf]) - cz
                tgt = (obase + (j * 16 + lanes) * 3)
                plsc.store_scatter(ob, [tgt], gx)
                plsc.store_scatter(ob, [tgt + 1], gy)
                plsc.store_scatter(ob, [tgt + 2], gz)
        return 0

    lax.fori_loop(0, SH, run_s, 0)

    for (nsamp, ib, ob), g_hbm in zip(scales, (g0_hbm, g1_hbm, g2_hbm)):
        off = (b * S + s0) * nsamp * 3
        pltpu.sync_copy(ob, g_hbm.at[pl.ds(off, SH * nsamp * 3)])


def _run_group(xt, cent, code):
    mesh = plsc.VectorSubcoreMesh(core_axis_name="c", subcore_axis_name="s")
    k = functools.partial(
        pl.kernel,
        mesh=mesh,
        compiler_params=pltpu.CompilerParams(needs_layout_passes=False),
        out_type=[
            jax.ShapeDtypeStruct((B * S * NSAMPLES[0] * 3,), jnp.float32),
            jax.ShapeDtypeStruct((B * S * NSAMPLES[1] * 3,), jnp.float32),
            jax.ShapeDtypeStruct((B * S * NSAMPLES[2] * 3,), jnp.float32),
        ],
        scratch_types=[
            pltpu.VMEM((N,), jnp.float32),
            pltpu.VMEM((N,), jnp.float32),
            pltpu.VMEM((N,), jnp.float32),
            pltpu.VMEM((3 * SH,), jnp.float32),
            pltpu.VMEM((N,), jnp.int32),
            pltpu.VMEM((256,), jnp.int32),
            pltpu.VMEM((256,), jnp.int32),
            pltpu.VMEM((256,), jnp.int32),
            pltpu.VMEM((SH * NSAMPLES[0] * 3,), jnp.float32),
            pltpu.VMEM((SH * NSAMPLES[1] * 3,), jnp.float32),
            pltpu.VMEM((SH * NSAMPLES[2] * 3,), jnp.float32),
        ],
    )(_group_body)
    return k(xt, cent, code)


def _mlp_body(G, ns, C3, g_ref, w1_ref, b1_ref, w2_ref, b2_ref, w3_ref,
              b3_ref, wp_ref, out_ref):
    g = g_ref[...]
    h = jnp.dot(g, w1_ref[...], preferred_element_type=jnp.float32) + b1_ref[...]
    h = jnp.maximum(h, 0.0)
    h = jnp.dot(h, w2_ref[...], preferred_element_type=jnp.float32) + b2_ref[...]
    h = jnp.maximum(h, 0.0)
    h = jnp.dot(h, w3_ref[...], preferred_element_type=jnp.float32) + b3_ref[...]
    h = jnp.maximum(h, 0.0)
    f = jnp.max(h.reshape(G, ns, C3), axis=1)
    out_ref[...] = jnp.dot(f, wp_ref[...], preferred_element_type=jnp.float32)


def _run_mlp(g, ns, G, w1t, b1, w2t, b2, w3t, b3, wp):
    R = B * S
    P = G * ns
    C1, C2, C3 = w1t.shape[1], w2t.shape[1], w3t.shape[1]
    body = functools.partial(_mlp_body, G, ns, C3)
    rep = lambda i: (0, 0)
    return pl.pallas_call(
        body,
        grid=(R // G,),
        in_specs=[
            pl.BlockSpec((P, 3), lambda i: (i, 0)),
            pl.BlockSpec((3, C1), rep),
            pl.BlockSpec((1, C1), rep),
            pl.BlockSpec((C1, C2), rep),
            pl.BlockSpec((1, C2), rep),
            pl.BlockSpec((C2, C3), rep),
            pl.BlockSpec((1, C3), rep),
            pl.BlockSpec((C3, 1), rep),
        ],
        out_specs=pl.BlockSpec((G, 1), lambda i: (i, 0)),
        out_shape=jax.ShapeDtypeStruct((R, 1), jnp.float32),
    )(g, w1t, b1, w2t, b2, w3t, b3, wp)


_DEBUG_XLA_GROUP = False


def _xla_group(xyz, cent, code):
    c = jnp.transpose(cent, (1, 2, 0))
    outs = []
    for k, ns in enumerate(NSAMPLES):
        mask = (code >> k) & 1
        gidx = jnp.broadcast_to(jnp.arange(N, dtype=jnp.int32), (B, S, N))
        gidx = jnp.where(mask == 0, N, gidx)
        gidx = jnp.sort(gidx, axis=-1)[:, :, :ns]
        first = gidx[:, :, :1]
        gidx = jnp.where(gidx == N, jnp.broadcast_to(first, gidx.shape), gidx)
        batch = jnp.arange(B).reshape(B, 1, 1)
        g = xyz[batch, gidx] - c[:, :, None, :]
        outs.append(g.reshape(-1))
    return tuple(outs)

def kernel(xyz, W0_0, b0_0, W0_1, b0_1, W0_2, b0_2, W1_0, b1_0, W1_1, b1_1,
           W1_2, b1_2, W2_0, b2_0, W2_1, b2_1, W2_2, b2_2, Wp, bp):
    xt = jnp.transpose(xyz, (2, 0, 1))
    cent = _run_fps(xt)
    code = _run_mask(jnp.transpose(cent, (1, 2, 0)), jnp.transpose(xyz, (0, 2, 1)))
    if _DEBUG_XLA_GROUP:
        g0, g1, g2 = _xla_group(xyz, cent, code)
    else:
        g0, g1, g2 = _run_group(xt, cent, code)

    weights = (
        (W0_0, b0_0, W0_1, b0_1, W0_2, b0_2),
        (W1_0, b1_0, W1_1, b1_1, W1_2, b1_2),
        (W2_0, b2_0, W2_1, b2_1, W2_2, b2_2),
    )
    c3s = (32, 64, 64)
    gs = (g0, g1, g2)
    Gs = (256, 128, 64)
    parts = []
    off = 0
    for k in range(3):
        w1, bb1, w2, bb2, w3, bb3 = weights[k]
        ns = NSAMPLES[k]
        c3 = c3s[k]
        wp_k = Wp[0, off:off + c3].reshape(c3, 1)
        off += c3
        g2d = gs[k].reshape(B * S * ns, 3)
        parts.append(_run_mlp(
            g2d, ns, Gs[k],
            w1.T, bb1.reshape(1, -1),
            w2.T, bb2.reshape(1, -1),
            w3.T, bb3.reshape(1, -1),
            wp_k))
    out = (parts[0] + parts[1] + parts[2]).reshape(B, 1, S) + bp[0]
    return out
